# baseline (speedup 1.0000x reference)
import functools

import jax
import jax.numpy as jnp
from jax import lax
from jax.experimental import pallas as pl
from jax.experimental.pallas import tpu as pltpu
from jax.experimental.pallas import tpu_sc as plsc

N = 100000
E = 3200000
NC = 2
NS = 16
NW = NC * NS
EPAD = 3276800
GCHUNK = 1024
GITER = EPAD // NW // GCHUNK
SCHUNK = 1024
SROWS_PER_CORE = EPAD // 128 // NC
SITER = SROWS_PER_CORE // NS // 8
NAGG = 100352

_mesh = plsc.VectorSubcoreMesh(core_axis_name="c", subcore_axis_name="s")
_sc_params = pltpu.CompilerParams(use_tc_tiling_on_sc=False)


@functools.partial(
    pl.kernel,
    out_type=(
        jax.ShapeDtypeStruct((EPAD, 16), jnp.bfloat16),
        jax.ShapeDtypeStruct((EPAD, 16), jnp.bfloat16),
    ),
    mesh=_mesh,
    scratch_types=[
        pltpu.VMEM((8, 128), jnp.int32),
        pltpu.VMEM((8, 128), jnp.int32),
        pltpu.VMEM((GCHUNK, 16), jnp.bfloat16),
        pltpu.VMEM((GCHUNK, 16), jnp.bfloat16),
        pltpu.SemaphoreType.DMA,
    ],
    compiler_params=_sc_params,
)
def _sc_gather(table_hbm, sidx_hbm, didx_hbm, srows_hbm, drows_hbm,
               idxs_v, idxd_v, bufs_v, bufd_v, sem):
    c = lax.axis_index("c")
    s = lax.axis_index("s")
    wid = s * NC + c

    def chunk(i, carry):
        base = pl.multiple_of(wid * (EPAD // NW) + i * GCHUNK, 1024)
        row0 = pl.multiple_of(base // 128, 8)
        pltpu.sync_copy(sidx_hbm.at[pl.ds(row0, 8)], idxs_v)
        pltpu.sync_copy(didx_hbm.at[pl.ds(row0, 8)], idxd_v)
        descs = []
        for j in range(8):
            descs.append(pltpu.async_copy(
                table_hbm.at[idxs_v.at[j]],
                bufs_v.at[pl.ds(j * 128, 128)], sem))
            descs.append(pltpu.async_copy(
                table_hbm.at[idxd_v.at[j]],
                bufd_v.at[pl.ds(j * 128, 128)], sem))
        for d in descs:
            d.wait()
        pltpu.sync_copy(bufs_v, srows_hbm.at[pl.ds(base, GCHUNK)])
        pltpu.sync_copy(bufd_v, drows_hbm.at[pl.ds(base, GCHUNK)])
        return carry

    lax.fori_loop(0, GITER, chunk, 0)


@functools.partial(
    pl.kernel,
    out_type=jax.ShapeDtypeStruct((NC, NAGG), jnp.float32),
    mesh=_mesh,
    scratch_types=[
        pltpu.VMEM((8, 128), jnp.int32),
        pltpu.VMEM((8, 128), jnp.float32),
        pltpu.VMEM((2048,), jnp.float32),
        pltpu.VMEM_SHARED((NAGG,), jnp.float32),
        pltpu.SemaphoreType.DMA,
    ],
    compiler_params=_sc_params,
)
def _sc_scatter(didx_hbm, m_hbm, agg_hbm, idx_v, val_v, zbuf_v, agg_sp, sem):
    c = lax.axis_index("c")
    s = lax.axis_index("s")

    @pl.when(s == 0)
    def _zero():
        def zb(k, carry):
            zbuf_v[pl.ds(k * 16, 16)] = jnp.zeros((16,), jnp.float32)
            return carry
        lax.fori_loop(0, 2048 // 16, zb, 0)

        def zs(k, carry):
            pltpu.sync_copy(zbuf_v, agg_sp.at[pl.ds(k * 2048, 2048)])
            return carry
        lax.fori_loop(0, NAGG // 2048, zs, 0)

    plsc.subcore_barrier()

    def chunk(i, carry):
        row0 = pl.multiple_of(
            c * SROWS_PER_CORE + s * (SROWS_PER_CORE // NS) + i * 8, 8)
        pltpu.sync_copy(didx_hbm.at[pl.ds(row0, 8)], idx_v)
        pltpu.sync_copy(m_hbm.at[pl.ds(row0, 8)], val_v)
        descs = []
        for j in range(8):
            descs.append(pltpu.async_copy(
                val_v.at[j], agg_sp.at[idx_v.at[j]], sem, add=True))
        for d in descs:
            d.wait()
        return carry

    lax.fori_loop(0, SITER, chunk, 0)

    plsc.subcore_barrier()

    @pl.when(s == 0)
    def _writeout():
        pltpu.sync_copy(agg_sp, agg_hbm.at[c])


def _phi_body(s_ref, d_ref, k0s_ref, k0d_ref, b0_ref, k1_ref, b1_ref,
              k2_ref, b2_ref, m_ref):
    bf = jnp.bfloat16
    sv = s_ref[...]
    dv = d_ref[...]
    h = jnp.tanh(
        jnp.dot(sv, k0s_ref[...], preferred_element_type=jnp.float32)
        + jnp.dot(dv, k0d_ref[...], preferred_element_type=jnp.float32)
        + b0_ref[...])
    h = jnp.tanh(
        jnp.dot(h.astype(bf), k1_ref[...], preferred_element_type=jnp.float32)
        + b1_ref[...])
    m_ref[...] = (
        jnp.dot(h.astype(bf), k2_ref[...], preferred_element_type=jnp.float32)
        + b2_ref[...])


def _run_phi(sp, dp, k0s, k0d, b0, k1, b1, k2, b2):
    rows = EPAD // 8
    blk = 8192
    grid = rows // blk
    full = lambda shape: pl.BlockSpec(shape, lambda i: (0, 0))
    return pl.pallas_call(
        _phi_body,
        grid=(grid,),
        in_specs=[
            pl.BlockSpec((blk, 128), lambda i: (i, 0)),
            pl.BlockSpec((blk, 128), lambda i: (i, 0)),
            full((128, 256)), full((128, 256)), full((1, 256)),
            full((256, 256)), full((1, 256)),
            full((256, 8)), full((1, 8)),
        ],
        out_specs=pl.BlockSpec((blk, 8), lambda i: (i, 0)),
        out_shape=jax.ShapeDtypeStruct((rows, 8), jnp.float32),
    )(sp, dp, k0s, k0d, b0, k1, b1, k2, b2)


def _gamma_body(x_ref, a0_ref, a1_ref, xl_ref, g0x_ref, g0a_ref, b0_ref,
                g1_ref, b1_ref, g2_ref, b2_ref, o_ref):
    a = a0_ref[...] + a1_ref[...]
    h = jnp.tanh(
        jnp.dot(x_ref[...], g0x_ref[...], preferred_element_type=jnp.float32)
        + jnp.dot(a, g0a_ref[...], preferred_element_type=jnp.float32)
        + b0_ref[...])
    h = jnp.tanh(
        jnp.dot(h, g1_ref[...], preferred_element_type=jnp.float32)
        + b1_ref[...])
    o_ref[...] = (xl_ref[...]
                  + jnp.dot(h, g2_ref[...], preferred_element_type=jnp.float32)
                  + b2_ref[...])


def _run_gamma(xp, a0, a1, xl, g0x, g0a, b0, g1, b1, g2, b2):
    rows = N // 16
    return pl.pallas_call(
        _gamma_body,
        out_shape=jax.ShapeDtypeStruct((rows, 16), jnp.float32),
    )(xp, a0, a1, xl, g0x, g0a, b0, g1, b1, g2, b2)


def kernel(x, pos, edge_index, pW0, pb0, pW1, pb1, pW2, pb2,
           gW0, gb0, gW1, gb1, gW2, gb2):
    f32 = jnp.float32
    table = jnp.concatenate(
        [x, pos, jnp.zeros((N, 6), f32)], axis=1).astype(jnp.bfloat16)

    src = edge_index[0]
    dst = edge_index[1]
    pad = EPAD - E
    src_p = jnp.concatenate([src, jnp.zeros((pad,), jnp.int32)])
    dst_p = jnp.concatenate([dst, jnp.full((pad,), N, jnp.int32)])
    sidx = src_p.reshape(EPAD // 128, 128)
    didx = dst_p.reshape(EPAD // 128, 128)

    srows, drows = _sc_gather(table, sidx, didx)

    i8 = jnp.eye(8, dtype=f32)
    ws = jnp.zeros((16, 32), f32).at[0:8].set(pW0[8:16]).at[8:10].set(pW0[16:18])
    wd = jnp.zeros((16, 32), f32).at[0:8].set(pW0[0:8]).at[8:10].set(-pW0[16:18])
    bf = jnp.bfloat16
    k0s = jnp.kron(i8, ws).astype(bf)
    k0d = jnp.kron(i8, wd).astype(bf)
    b0 = jnp.tile(pb0, 8).reshape(1, 256)
    k1 = jnp.kron(i8, pW1).astype(bf)
    b1 = jnp.tile(pb1, 8).reshape(1, 256)
    k2 = jnp.kron(i8, pW2).astype(bf)
    b2 = jnp.tile(pb2, 8).reshape(1, 8)

    sp = srows.reshape(EPAD // 8, 128)
    dp = drows.reshape(EPAD // 8, 128)
    m2d = _run_phi(sp, dp, k0s, k0d, b0, k1, b1, k2, b2)

    mrows = m2d.reshape(EPAD // 128, 128)
    agg2 = _sc_scatter(didx, mrows)

    i16 = jnp.eye(16, dtype=f32)
    g0x = jnp.kron(i16, gW0[0:8])
    g0a = jnp.kron(i16, gW0[8:9])
    gb0t = jnp.tile(gb0, 16).reshape(1, 512)
    g1 = jnp.kron(i16, gW1)
    gb1t = jnp.tile(gb1, 16).reshape(1, 512)
    g2 = jnp.kron(i16, gW2)
    gb2t = jnp.tile(gb2, 16).reshape(1, 16)

    xp = x.reshape(N // 16, 128)
    a0 = agg2[0, :N].reshape(N // 16, 16)
    a1 = agg2[1, :N].reshape(N // 16, 16)
    xl = x[:, 7].reshape(N // 16, 16)

    out = _run_gamma(xp, a0, a1, xl, g0x, g0a, gb0t, g1, gb1t, g2, gb2t)
    return out.reshape(N, 1)


# speedup vs baseline: 1.3801x; 1.3801x over previous
import functools

import jax
import jax.numpy as jnp
from jax import lax
from jax.experimental import pallas as pl
from jax.experimental.pallas import tpu as pltpu
from jax.experimental.pallas import tpu_sc as plsc

N = 100000
E = 3200000
NC = 2
NS = 16
NW = NC * NS
EPAD = 3276800
GCHUNK = 1024
GITER = EPAD // NW // GCHUNK
SCHUNK = 1024
SROWS_PER_CORE = EPAD // 128 // NC
SITER = SROWS_PER_CORE // NS // 8
NAGG = 100352

_mesh = plsc.VectorSubcoreMesh(core_axis_name="c", subcore_axis_name="s")
_sc_params = pltpu.CompilerParams(use_tc_tiling_on_sc=False)


@functools.partial(
    pl.kernel,
    out_type=(
        jax.ShapeDtypeStruct((EPAD, 16), jnp.float32),
        jax.ShapeDtypeStruct((EPAD, 16), jnp.float32),
    ),
    mesh=_mesh,
    scratch_types=[
        pltpu.VMEM((2, 8, 128), jnp.int32),
        pltpu.VMEM((2, 8, 128), jnp.int32),
        pltpu.VMEM((2, GCHUNK, 16), jnp.float32),
        pltpu.VMEM((2, GCHUNK, 16), jnp.float32),
        pltpu.SemaphoreType.DMA,
        pltpu.SemaphoreType.DMA,
        pltpu.SemaphoreType.DMA,
    ],
    compiler_params=_sc_params,
)
def _sc_gather(table_hbm, sidx_hbm, didx_hbm, srows_hbm, drows_hbm,
               idxs_v, idxd_v, bufs_v, bufd_v, gsem, wsem0, wsem1):
    c = lax.axis_index("c")
    s = lax.axis_index("s")
    wid = s * NC + c
    wsems = (wsem0, wsem1)

    def outer(g, carry):
        for p in range(2):
            i = g * 2 + p
            base = pl.multiple_of(wid * (EPAD // NW) + i * GCHUNK, 1024)
            row0 = pl.multiple_of(base // 128, 8)

            @pl.when(g > 0)
            def _drain():
                pltpu.make_async_copy(
                    bufs_v.at[p], srows_hbm.at[pl.ds(base, GCHUNK)],
                    wsems[p]).wait()
                pltpu.make_async_copy(
                    bufd_v.at[p], drows_hbm.at[pl.ds(base, GCHUNK)],
                    wsems[p]).wait()

            pltpu.sync_copy(sidx_hbm.at[pl.ds(row0, 8)], idxs_v.at[p])
            pltpu.sync_copy(didx_hbm.at[pl.ds(row0, 8)], idxd_v.at[p])
            descs = []
            for j in range(8):
                descs.append(pltpu.async_copy(
                    table_hbm.at[idxs_v.at[p].at[j]],
                    bufs_v.at[p].at[pl.ds(j * 128, 128)], gsem))
                descs.append(pltpu.async_copy(
                    table_hbm.at[idxd_v.at[p].at[j]],
                    bufd_v.at[p].at[pl.ds(j * 128, 128)], gsem))
            for d in descs:
                d.wait()
            pltpu.async_copy(
                bufs_v.at[p], srows_hbm.at[pl.ds(base, GCHUNK)], wsems[p])
            pltpu.async_copy(
                bufd_v.at[p], drows_hbm.at[pl.ds(base, GCHUNK)], wsems[p])
        return carry

    lax.fori_loop(0, GITER // 2, outer, 0)

    for p in range(2):
        base = pl.multiple_of(
            wid * (EPAD // NW) + (GITER - 2 + p) * GCHUNK, 1024)
        pltpu.make_async_copy(
            bufs_v.at[p], srows_hbm.at[pl.ds(base, GCHUNK)], wsems[p]).wait()
        pltpu.make_async_copy(
            bufd_v.at[p], drows_hbm.at[pl.ds(base, GCHUNK)], wsems[p]).wait()


@functools.partial(
    pl.kernel,
    out_type=jax.ShapeDtypeStruct((NC, NAGG), jnp.float32),
    mesh=_mesh,
    scratch_types=[
        pltpu.VMEM((8, 128), jnp.int32),
        pltpu.VMEM((8, 128), jnp.float32),
        pltpu.VMEM((2048,), jnp.float32),
        pltpu.VMEM_SHARED((NAGG,), jnp.float32),
        pltpu.SemaphoreType.DMA,
    ],
    compiler_params=_sc_params,
)
def _sc_scatter(didx_hbm, m_hbm, agg_hbm, idx_v, val_v, zbuf_v, agg_sp, sem):
    c = lax.axis_index("c")
    s = lax.axis_index("s")

    @pl.when(s == 0)
    def _zero():
        def zb(k, carry):
            zbuf_v[pl.ds(k * 16, 16)] = jnp.zeros((16,), jnp.float32)
            return carry
        lax.fori_loop(0, 2048 // 16, zb, 0)

        def zs(k, carry):
            pltpu.sync_copy(zbuf_v, agg_sp.at[pl.ds(k * 2048, 2048)])
            return carry
        lax.fori_loop(0, NAGG // 2048, zs, 0)

    plsc.subcore_barrier()

    def chunk(i, carry):
        row0 = pl.multiple_of(
            c * SROWS_PER_CORE + s * (SROWS_PER_CORE // NS) + i * 8, 8)
        pltpu.sync_copy(didx_hbm.at[pl.ds(row0, 8)], idx_v)
        pltpu.sync_copy(m_hbm.at[pl.ds(row0, 8)], val_v)
        descs = []
        for j in range(8):
            descs.append(pltpu.async_copy(
                val_v.at[j], agg_sp.at[idx_v.at[j]], sem, add=True))
        for d in descs:
            d.wait()
        return carry

    lax.fori_loop(0, SITER, chunk, 0)

    plsc.subcore_barrier()

    @pl.when(s == 0)
    def _writeout():
        pltpu.sync_copy(agg_sp, agg_hbm.at[c])


def _phi_body(s_ref, d_ref, k0s_ref, k0d_ref, b0_ref, k1_ref, b1_ref,
              k2_ref, b2_ref, m_ref):
    h = jnp.tanh(
        jnp.dot(s_ref[...], k0s_ref[...], preferred_element_type=jnp.float32)
        + jnp.dot(d_ref[...], k0d_ref[...], preferred_element_type=jnp.float32)
        + b0_ref[...])
    h = jnp.tanh(
        jnp.dot(h, k1_ref[...], preferred_element_type=jnp.float32)
        + b1_ref[...])
    m_ref[...] = (
        jnp.dot(h, k2_ref[...], preferred_element_type=jnp.float32)
        + b2_ref[...])


def _run_phi(sp, dp, k0s, k0d, b0, k1, b1, k2, b2):
    rows = EPAD // 8
    blk = 8192
    grid = rows // blk
    full = lambda shape: pl.BlockSpec(shape, lambda i: (0, 0))
    return pl.pallas_call(
        _phi_body,
        grid=(grid,),
        in_specs=[
            pl.BlockSpec((blk, 128), lambda i: (i, 0)),
            pl.BlockSpec((blk, 128), lambda i: (i, 0)),
            full((128, 256)), full((128, 256)), full((1, 256)),
            full((256, 256)), full((1, 256)),
            full((256, 8)), full((1, 8)),
        ],
        out_specs=pl.BlockSpec((blk, 8), lambda i: (i, 0)),
        out_shape=jax.ShapeDtypeStruct((rows, 8), jnp.float32),
    )(sp, dp, k0s, k0d, b0, k1, b1, k2, b2)


def _gamma_body(x_ref, a0_ref, a1_ref, xl_ref, g0x_ref, g0a_ref, b0_ref,
                g1_ref, b1_ref, g2_ref, b2_ref, o_ref):
    a = a0_ref[...] + a1_ref[...]
    h = jnp.tanh(
        jnp.dot(x_ref[...], g0x_ref[...], preferred_element_type=jnp.float32)
        + jnp.dot(a, g0a_ref[...], preferred_element_type=jnp.float32)
        + b0_ref[...])
    h = jnp.tanh(
        jnp.dot(h, g1_ref[...], preferred_element_type=jnp.float32)
        + b1_ref[...])
    o_ref[...] = (xl_ref[...]
                  + jnp.dot(h, g2_ref[...], preferred_element_type=jnp.float32)
                  + b2_ref[...])


def _run_gamma(xp, a0, a1, xl, g0x, g0a, b0, g1, b1, g2, b2):
    rows = N // 16
    return pl.pallas_call(
        _gamma_body,
        out_shape=jax.ShapeDtypeStruct((rows, 16), jnp.float32),
    )(xp, a0, a1, xl, g0x, g0a, b0, g1, b1, g2, b2)


def kernel(x, pos, edge_index, pW0, pb0, pW1, pb1, pW2, pb2,
           gW0, gb0, gW1, gb1, gW2, gb2):
    f32 = jnp.float32
    table = jnp.concatenate(
        [x, pos, jnp.zeros((N, 6), f32)], axis=1)

    src = edge_index[0]
    dst = edge_index[1]
    pad = EPAD - E
    src_p = jnp.concatenate([src, jnp.zeros((pad,), jnp.int32)])
    dst_p = jnp.concatenate([dst, jnp.full((pad,), N, jnp.int32)])
    sidx = src_p.reshape(EPAD // 128, 128)
    didx = dst_p.reshape(EPAD // 128, 128)

    srows, drows = _sc_gather(table, sidx, didx)

    i8 = jnp.eye(8, dtype=f32)
    ws = jnp.zeros((16, 32), f32).at[0:8].set(pW0[8:16]).at[8:10].set(pW0[16:18])
    wd = jnp.zeros((16, 32), f32).at[0:8].set(pW0[0:8]).at[8:10].set(-pW0[16:18])
    k0s = jnp.kron(i8, ws)
    k0d = jnp.kron(i8, wd)
    b0 = jnp.tile(pb0, 8).reshape(1, 256)
    k1 = jnp.kron(i8, pW1)
    b1 = jnp.tile(pb1, 8).reshape(1, 256)
    k2 = jnp.kron(i8, pW2)
    b2 = jnp.tile(pb2, 8).reshape(1, 8)

    sp = srows.reshape(EPAD // 8, 128)
    dp = drows.reshape(EPAD // 8, 128)
    m2d = _run_phi(sp, dp, k0s, k0d, b0, k1, b1, k2, b2)

    mrows = m2d.reshape(EPAD // 128, 128)
    agg2 = _sc_scatter(didx, mrows)

    i16 = jnp.eye(16, dtype=f32)
    g0x = jnp.kron(i16, gW0[0:8])
    g0a = jnp.kron(i16, gW0[8:9])
    gb0t = jnp.tile(gb0, 16).reshape(1, 512)
    g1 = jnp.kron(i16, gW1)
    gb1t = jnp.tile(gb1, 16).reshape(1, 512)
    g2 = jnp.kron(i16, gW2)
    gb2t = jnp.tile(gb2, 16).reshape(1, 16)

    xp = x.reshape(N // 16, 128)
    a0 = agg2[0, :N].reshape(N // 16, 16)
    a1 = agg2[1, :N].reshape(N // 16, 16)
    xl = x[:, 7].reshape(N // 16, 16)

    out = _run_gamma(xp, a0, a1, xl, g0x, g0a, gb0t, g1, gb1t, g2, gb2t)
    return out.reshape(N, 1)
